# baseline (speedup 1.0000x reference)
import functools

import jax
import jax.numpy as jnp
from jax import lax
from jax.experimental import pallas as pl
from jax.experimental.pallas import tpu as pltpu
from jax.experimental.pallas import tpu_sc as plsc

D = 96
C = 8
B = 2
PLANE = D * D
NROWS = B * D * D
NVOX = B * D * D * D
NW = 32
PPW = (B * D) // NW
GPR = D // 16
NIDX = 8 * D
NCHUNK = NIDX // 128


def _sc_body(vol_hbm, trf_hbm, out_hbm, trf_v, idx_v, w_v, rows_v, out_v, sem):
    wid = lax.axis_index("s") * 2 + lax.axis_index("c")
    lane = lax.iota(jnp.int32, 16)
    lane_f = lane.astype(jnp.float32)

    def plane_body(dp, _):
        p = wid * PPW + dp
        b = jnp.where(p >= D, 1, 0)
        i = p - D * b
        i_f = i.astype(jnp.float32)
        base_row = p * PLANE

        def row_body(j, _):
            vb = base_row + j * D
            j_f = j.astype(jnp.float32)
            pltpu.sync_copy(trf_hbm.at[pl.ds(vb, D)], trf_v)

            for g in range(GPR):
                kv = jnp.full((16,), 16 * g, jnp.int32) + lane
                tx = plsc.load_gather(trf_v, [kv, jnp.full((16,), 0, jnp.int32)])
                ty = plsc.load_gather(trf_v, [kv, jnp.full((16,), 1, jnp.int32)])
                tz = plsc.load_gather(trf_v, [kv, jnp.full((16,), 2, jnp.int32)])
                ci = jnp.minimum(jnp.maximum(i_f + tx, 0.0), float(D - 1))
                cj = jnp.minimum(jnp.maximum(j_f + ty, 0.0), float(D - 1))
                ck = jnp.minimum(jnp.maximum(
                    (16.0 * g) + lane_f + tz, 0.0), float(D - 1))
                bi = jnp.minimum(ci.astype(jnp.int32), D - 2)
                bj = jnp.minimum(cj.astype(jnp.int32), D - 2)
                bk = jnp.minimum(ck.astype(jnp.int32), D - 2)
                wi1 = ci - bi.astype(jnp.float32)
                wj1 = cj - bj.astype(jnp.float32)
                wk1 = ck - bk.astype(jnp.float32)
                wi0 = 1.0 - wi1
                wj0 = 1.0 - wj1
                wk0 = 1.0 - wk1
                rb = b * (D * PLANE) + bi * PLANE + bj * D + bk
                wlist = (wi0 * wj0, wi0 * wj1, wi1 * wj0, wi1 * wj1)
                for c4, (di, dj) in enumerate(((0, 0), (0, 1), (1, 0), (1, 1))):
                    wij = wlist[c4]
                    for dk in (0, 1):
                        c8 = c4 * 2 + dk
                        pos = D * c8 + 16 * g
                        idx_v[pos // 128, pl.ds(pos % 128, 16)] = (
                            rb + (di * PLANE + dj * D + dk))
                        w_v[c8, pl.ds(16 * g, 16)] = wij * (wk1 if dk else wk0)

            copies = [
                pltpu.async_copy(
                    vol_hbm.at[idx_v.at[c]],
                    rows_v.at[pl.ds(128 * c, 128)],
                    sem,
                )
                for c in range(NCHUNK)
            ]
            for cp in copies:
                cp.wait()

            for g in range(GPR):
                rvec = [jnp.full((16,), D * c8 + 16 * g, jnp.int32) + lane
                        for c8 in range(8)]
                wv = [w_v[c8, pl.ds(16 * g, 16)] for c8 in range(8)]
                ovec = jnp.full((16,), 16 * g, jnp.int32) + lane
                for ch in range(C):
                    chv = jnp.full((16,), ch, jnp.int32)
                    acc = wv[0] * plsc.load_gather(rows_v, [rvec[0], chv])
                    for c8 in range(1, 8):
                        acc = acc + wv[c8] * plsc.load_gather(
                            rows_v, [rvec[c8], chv])
                    plsc.store_scatter(out_v, [ovec, chv], acc)

            pltpu.sync_copy(out_v, out_hbm.at[pl.ds(vb, D)])
            return 0

        lax.fori_loop(0, D, row_body, 0)
        return 0

    lax.fori_loop(0, PPW, plane_body, 0)


@jax.jit
def _spatial_transform(vol_flat, trf_flat):
    mesh = plsc.VectorSubcoreMesh(core_axis_name="c", subcore_axis_name="s")
    run = pl.kernel(
        _sc_body,
        out_type=jax.ShapeDtypeStruct((NVOX, C), jnp.float32),
        mesh=mesh,
        scratch_types=[
            pltpu.VMEM((D, 3), jnp.float32),
            pltpu.VMEM((NCHUNK, 128), jnp.int32),
            pltpu.VMEM((8, D), jnp.float32),
            pltpu.VMEM((NIDX, C), jnp.float32),
            pltpu.VMEM((D, C), jnp.float32),
            pltpu.SemaphoreType.DMA,
        ],
        compiler_params=pltpu.CompilerParams(
            needs_layout_passes=False,
            use_tc_tiling_on_sc=False,
        ),
    )
    return run(vol_flat, trf_flat)


def kernel(vol, trf):
    vol_flat = vol.reshape(NVOX, C)
    trf_flat = trf.reshape(NVOX, 3)
    out = _spatial_transform(vol_flat, trf_flat)
    return out.reshape(B, D, D, D, C)


# speedup vs baseline: 1.1365x; 1.1365x over previous
import jax
import jax.numpy as jnp
from jax import lax
from jax.experimental import pallas as pl
from jax.experimental.pallas import tpu as pltpu
from jax.experimental.pallas import tpu_sc as plsc

D = 96
C = 8
B = 2
PLANE = D * D
NVOX = B * D * D * D
NW = 32
PPW = (B * D) // NW
VPW = NVOX // NW
CH = 6
VC = CH * D
NCHK = VPW // VC
NIDX = 8 * VC
NLIST = NIDX // 128
GPR = D // 16


def _sc_body(vol_hbm, trf_hbm, out_hbm,
             trf_a, trf_b, idx_a, idx_b, w_a, w_b, rows_a, rows_b,
             out_a, out_b,
             trf_sem_a, trf_sem_b, gat_sem_a, gat_sem_b,
             out_sem_a, out_sem_b):
    wid = lax.axis_index("s") * 2 + lax.axis_index("c")
    tile_base = wid * VPW
    lane = lax.iota(jnp.int32, 16)
    lane_f = lane.astype(jnp.float32)

    def make_prepare(trf_r, idx_r, rows_r, w_r, trf_sem, gat_sem):
        def prepare(x, jp, dpp):
            base = tile_base + x * VC
            pltpu.make_async_copy(
                trf_hbm.at[pl.ds(base, VC)], trf_r, trf_sem).wait()
            plane = wid * PPW + dpp
            b = jnp.where(plane >= D, 1, 0)
            i_f = (plane - D * b).astype(jnp.float32)
            vol_b_off = b * (D * PLANE)

            def row(r, _):
                j_f = (jp + r).astype(jnp.float32)
                voff0 = D * r
                for gr in range(GPR):
                    voff = voff0 + 16 * gr
                    vvec = jnp.full((16,), voff, jnp.int32) + lane
                    tx = plsc.load_gather(trf_r, [vvec, jnp.full((16,), 0, jnp.int32)])
                    ty = plsc.load_gather(trf_r, [vvec, jnp.full((16,), 1, jnp.int32)])
                    tz = plsc.load_gather(trf_r, [vvec, jnp.full((16,), 2, jnp.int32)])
                    ci = jnp.minimum(jnp.maximum(i_f + tx, 0.0), float(D - 1))
                    cj = jnp.minimum(jnp.maximum(j_f + ty, 0.0), float(D - 1))
                    ck = jnp.minimum(jnp.maximum(
                        (16.0 * gr) + lane_f + tz, 0.0), float(D - 1))
                    bi = jnp.minimum(ci.astype(jnp.int32), D - 2)
                    bj = jnp.minimum(cj.astype(jnp.int32), D - 2)
                    bk = jnp.minimum(ck.astype(jnp.int32), D - 2)
                    wi1 = ci - bi.astype(jnp.float32)
                    wj1 = cj - bj.astype(jnp.float32)
                    wk1 = ck - bk.astype(jnp.float32)
                    wi0 = 1.0 - wi1
                    wj0 = 1.0 - wj1
                    wk0 = 1.0 - wk1
                    rb = vol_b_off + bi * PLANE + bj * D + bk
                    wlist = (wi0 * wj0, wi0 * wj1, wi1 * wj0, wi1 * wj1)
                    ibase = 8 * voff0 + 16 * gr
                    for c4, (di, dj) in enumerate(
                            ((0, 0), (0, 1), (1, 0), (1, 1))):
                        for dk in (0, 1):
                            c8 = c4 * 2 + dk
                            idx_r[pl.ds(ibase + D * c8, 16)] = (
                                rb + (di * PLANE + dj * D + dk))
                            w_r[c8, pl.ds(voff, 16)] = (
                                wlist[c4] * (wk1 if dk else wk0))
                for lr in range(GPR):
                    off = 8 * voff0 + 128 * lr
                    pltpu.async_copy(
                        vol_hbm.at[idx_r.at[pl.ds(off, 128)]],
                        rows_r.at[pl.ds(off, 128)], gat_sem)
                return 0

            lax.fori_loop(0, CH, row, 0)

            @pl.when(x + 2 < NCHK)
            def _():
                pltpu.async_copy(
                    trf_hbm.at[pl.ds(tile_base + (x + 2) * VC, VC)],
                    trf_r, trf_sem)
        return prepare

    def make_reduce(idx_r, rows_r, w_r, out_r, gat_sem, out_sem):
        def reduce(c):
            @pl.when(c >= 2)
            def _():
                pltpu.make_async_copy(
                    out_r, out_hbm.at[pl.ds(0, VC)], out_sem).wait()

            def dr(l, _):
                off = 128 * l
                pltpu.make_async_copy(
                    vol_hbm.at[idx_r.at[pl.ds(off, 128)]],
                    rows_r.at[pl.ds(off, 128)], gat_sem).wait()
                return 0
            lax.fori_loop(0, NLIST, dr, 0)

            def row(r, _):
                voff0 = D * r
                for gr in range(GPR):
                    voff = voff0 + 16 * gr
                    ibase = 8 * voff0 + 16 * gr
                    rvecs = [jnp.full((16,), ibase + D * c8, jnp.int32) + lane
                             for c8 in range(8)]
                    wv = [w_r[c8, pl.ds(voff, 16)] for c8 in range(8)]
                    ovec = jnp.full((16,), voff, jnp.int32) + lane
                    for ch in range(C):
                        chv = jnp.full((16,), ch, jnp.int32)
                        acc = wv[0] * plsc.load_gather(rows_r, [rvecs[0], chv])
                        for c8 in range(1, 8):
                            acc = acc + wv[c8] * plsc.load_gather(
                                rows_r, [rvecs[c8], chv])
                        plsc.store_scatter(out_r, [ovec, chv], acc)
                return 0
            lax.fori_loop(0, CH, row, 0)

            pltpu.async_copy(
                out_r, out_hbm.at[pl.ds(tile_base + c * VC, VC)], out_sem)
        return reduce

    prepare_a = make_prepare(trf_a, idx_a, rows_a, w_a, trf_sem_a, gat_sem_a)
    prepare_b = make_prepare(trf_b, idx_b, rows_b, w_b, trf_sem_b, gat_sem_b)
    reduce_a = make_reduce(idx_a, rows_a, w_a, out_a, gat_sem_a, out_sem_a)
    reduce_b = make_reduce(idx_b, rows_b, w_b, out_b, gat_sem_b, out_sem_b)

    def advance(jp, dpp):
        jp2 = jp + CH
        wrap = jp2 >= D
        return jnp.where(wrap, jp2 - D, jp2), dpp + wrap.astype(jnp.int32)

    pltpu.async_copy(trf_hbm.at[pl.ds(tile_base, VC)], trf_a, trf_sem_a)
    pltpu.async_copy(trf_hbm.at[pl.ds(tile_base + VC, VC)], trf_b, trf_sem_b)
    z = jnp.int32(0)
    prepare_a(z, z, z)
    carry0 = advance(z, z)

    def t_loop(t, carry):
        jp, dpp = carry
        c0 = 2 * t
        prepare_b(c0 + 1, jp, dpp)
        jp, dpp = advance(jp, dpp)
        reduce_a(c0)

        @pl.when(c0 + 2 < NCHK)
        def _():
            prepare_a(c0 + 2, jp, dpp)
        carry = advance(jp, dpp)
        reduce_b(c0 + 1)
        return carry

    lax.fori_loop(0, NCHK // 2, t_loop, carry0)

    pltpu.make_async_copy(out_a, out_hbm.at[pl.ds(0, VC)], out_sem_a).wait()
    pltpu.make_async_copy(out_b, out_hbm.at[pl.ds(0, VC)], out_sem_b).wait()


@jax.jit
def _spatial_transform(vol_flat, trf_flat):
    mesh = plsc.VectorSubcoreMesh(core_axis_name="c", subcore_axis_name="s")
    run = pl.kernel(
        _sc_body,
        out_type=jax.ShapeDtypeStruct((NVOX, C), jnp.float32),
        mesh=mesh,
        scratch_types=[
            pltpu.VMEM((VC, 3), jnp.float32),
            pltpu.VMEM((VC, 3), jnp.float32),
            pltpu.VMEM((NIDX,), jnp.int32),
            pltpu.VMEM((NIDX,), jnp.int32),
            pltpu.VMEM((8, VC), jnp.float32),
            pltpu.VMEM((8, VC), jnp.float32),
            pltpu.VMEM((NIDX, C), jnp.float32),
            pltpu.VMEM((NIDX, C), jnp.float32),
            pltpu.VMEM((VC, C), jnp.float32),
            pltpu.VMEM((VC, C), jnp.float32),
            pltpu.SemaphoreType.DMA,
            pltpu.SemaphoreType.DMA,
            pltpu.SemaphoreType.DMA,
            pltpu.SemaphoreType.DMA,
            pltpu.SemaphoreType.DMA,
            pltpu.SemaphoreType.DMA,
        ],
        compiler_params=pltpu.CompilerParams(
            needs_layout_passes=False,
            use_tc_tiling_on_sc=False,
        ),
    )
    return run(vol_flat, trf_flat)


def kernel(vol, trf):
    vol_flat = vol.reshape(NVOX, C)
    trf_flat = trf.reshape(NVOX, 3)
    out = _spatial_transform(vol_flat, trf_flat)
    return out.reshape(B, D, D, D, C)


# speedup vs baseline: 3.7299x; 3.2820x over previous
import functools

import jax
import jax.numpy as jnp
from jax import lax
from jax.experimental import pallas as pl
from jax.experimental.pallas import tpu as pltpu
from jax.experimental.pallas import tpu_sc as plsc

D = 96
C = 8
B = 2
PLANE = D * D
NVOX = B * D * D * D
NCOL = B * D * D
NW = 32
PPW = (B * D) // NW
VPW = NVOX // NW
CH = 6
VC = CH * D
NCHK = VPW // VC
NIDX = 4 * VC
NLIST = NIDX // 128
GPR = D // 16
RB = 128


def _vol_pair_body(x_ref, o_ref):
    x = x_ref[...]
    xt = jnp.transpose(x, (0, 2, 1))
    xs = jnp.concatenate([xt[:, 1:, :], xt[:, D - 1:D, :]], axis=1)
    y = jnp.concatenate([xt, xs], axis=2)
    o_ref[...] = y.reshape(RB, D * 16)


def _out_back_body(x_ref, o_ref):
    x = x_ref[...].reshape(RB, D, C)
    o_ref[...] = jnp.transpose(x, (0, 2, 1))


def _sc_body(vol_hbm, tx_hbm, ty_hbm, tz_hbm, out_hbm,
             tx_a, ty_a, tz_a, tx_b, ty_b, tz_b,
             idx_a, idx_b, w_a, w_b, rows_a, rows_b, out_a, out_b,
             trf_sem_a, trf_sem_b, gat_sem_a, gat_sem_b,
             out_sem_a, out_sem_b):
    wid = lax.axis_index("s") * 2 + lax.axis_index("c")
    tile_base = wid * VPW
    lane = lax.iota(jnp.int32, 16)
    lane_f = lane.astype(jnp.float32)
    lane8 = lane * 8

    def make_prepare(tx_r, ty_r, tz_r, idx_r, rows_r, w_r, trf_sem, gat_sem):
        def prepare(x, jp, dpp):
            base = tile_base + x * VC
            for t_hbm, t_r in ((tx_hbm, tx_r), (ty_hbm, ty_r), (tz_hbm, tz_r)):
                pltpu.make_async_copy(
                    t_hbm.at[pl.ds(base, VC)], t_r, trf_sem).wait()
            plane = wid * PPW + dpp
            b = jnp.where(plane >= D, 1, 0)
            i_f = (plane - D * b).astype(jnp.float32)
            vol_b_off = b * (D * PLANE)

            def row(r, _):
                j_f = (jp + r).astype(jnp.float32)
                voff0 = D * r
                for gr in range(GPR):
                    voff = voff0 + 16 * gr
                    tx = tx_r[pl.ds(voff, 16)]
                    ty = ty_r[pl.ds(voff, 16)]
                    tz = tz_r[pl.ds(voff, 16)]
                    ci = jnp.minimum(jnp.maximum(i_f + tx, 0.0), float(D - 1))
                    cj = jnp.minimum(jnp.maximum(j_f + ty, 0.0), float(D - 1))
                    ck = jnp.minimum(jnp.maximum(
                        (16.0 * gr) + lane_f + tz, 0.0), float(D - 1))
                    bi = jnp.minimum(ci.astype(jnp.int32), D - 2)
                    bj = jnp.minimum(cj.astype(jnp.int32), D - 2)
                    bk = jnp.minimum(ck.astype(jnp.int32), D - 2)
                    wi1 = ci - bi.astype(jnp.float32)
                    wj1 = cj - bj.astype(jnp.float32)
                    wk1 = ck - bk.astype(jnp.float32)
                    wi0 = 1.0 - wi1
                    wj0 = 1.0 - wj1
                    wk0 = 1.0 - wk1
                    rb = vol_b_off + bi * PLANE + bj * D + bk
                    wlist = (wi0 * wj0, wi0 * wj1, wi1 * wj0, wi1 * wj1)
                    ibase = 4 * voff0 + 16 * gr
                    for cp, (di, dj) in enumerate(
                            ((0, 0), (0, 1), (1, 0), (1, 1))):
                        idx_r[pl.ds(ibase + D * cp, 16)] = (
                            rb + (di * PLANE + dj * D))
                        w_r[cp, pl.ds(voff, 16)] = wlist[cp] * wk0
                        w_r[cp + 4, pl.ds(voff, 16)] = wlist[cp] * wk1
                for lr in range(3):
                    off = 4 * voff0 + 128 * lr
                    pltpu.async_copy(
                        vol_hbm.at[idx_r.at[pl.ds(off, 128)]],
                        rows_r.at[pl.ds(off, 128)], gat_sem)
                return 0

            lax.fori_loop(0, CH, row, 0)

            @pl.when(x + 2 < NCHK)
            def _():
                nbase = tile_base + (x + 2) * VC
                for t_hbm, t_r in ((tx_hbm, tx_r), (ty_hbm, ty_r),
                                   (tz_hbm, tz_r)):
                    pltpu.async_copy(t_hbm.at[pl.ds(nbase, VC)], t_r, trf_sem)
        return prepare

    def make_reduce(idx_r, rows_r, w_r, out_r, gat_sem, out_sem):
        def reduce(c):
            @pl.when(c >= 2)
            def _():
                pltpu.make_async_copy(
                    out_r, out_hbm.at[pl.ds(0, VC * C)], out_sem).wait()

            def dr(l, _):
                off = 128 * l
                pltpu.make_async_copy(
                    vol_hbm.at[idx_r.at[pl.ds(off, 128)]],
                    rows_r.at[pl.ds(off, 128)], gat_sem).wait()
                return 0
            lax.fori_loop(0, NLIST, dr, 0)

            def row(r, _):
                voff0 = D * r
                for gr in range(GPR):
                    voff = voff0 + 16 * gr
                    ibase = 4 * voff0 + 16 * gr
                    rvecs = [jnp.full((16,), ibase + D * cp, jnp.int32) + lane
                             for cp in range(4)]
                    wv = [w_r[cp, pl.ds(voff, 16)] for cp in range(8)]
                    ovec = jnp.full((16,), 8 * voff, jnp.int32) + lane8
                    for ch in range(C):
                        chv = jnp.full((16,), ch, jnp.int32)
                        chv8 = jnp.full((16,), ch + 8, jnp.int32)
                        acc = wv[0] * plsc.load_gather(rows_r, [rvecs[0], chv])
                        for cp in range(1, 4):
                            acc = acc + wv[cp] * plsc.load_gather(
                                rows_r, [rvecs[cp], chv])
                        for cp in range(4):
                            acc = acc + wv[cp + 4] * plsc.load_gather(
                                rows_r, [rvecs[cp], chv8])
                        plsc.store_scatter(out_r, [ovec + ch], acc)
                return 0
            lax.fori_loop(0, CH, row, 0)

            pltpu.async_copy(
                out_r, out_hbm.at[pl.ds((tile_base + c * VC) * C, VC * C)],
                out_sem)
        return reduce

    prepare_a = make_prepare(tx_a, ty_a, tz_a, idx_a, rows_a, w_a,
                             trf_sem_a, gat_sem_a)
    prepare_b = make_prepare(tx_b, ty_b, tz_b, idx_b, rows_b, w_b,
                             trf_sem_b, gat_sem_b)
    reduce_a = make_reduce(idx_a, rows_a, w_a, out_a, gat_sem_a, out_sem_a)
    reduce_b = make_reduce(idx_b, rows_b, w_b, out_b, gat_sem_b, out_sem_b)

    def advance(jp, dpp):
        jp2 = jp + CH
        wrap = jp2 >= D
        return jnp.where(wrap, jp2 - D, jp2), dpp + wrap.astype(jnp.int32)

    for t_hbm, t_r in ((tx_hbm, tx_a), (ty_hbm, ty_a), (tz_hbm, tz_a)):
        pltpu.async_copy(t_hbm.at[pl.ds(tile_base, VC)], t_r, trf_sem_a)
    for t_hbm, t_r in ((tx_hbm, tx_b), (ty_hbm, ty_b), (tz_hbm, tz_b)):
        pltpu.async_copy(t_hbm.at[pl.ds(tile_base + VC, VC)], t_r, trf_sem_b)
    z = jnp.int32(0)
    prepare_a(z, z, z)
    carry0 = advance(z, z)

    def t_loop(t, carry):
        jp, dpp = carry
        c0 = 2 * t
        prepare_b(c0 + 1, jp, dpp)
        jp, dpp = advance(jp, dpp)
        reduce_a(c0)

        @pl.when(c0 + 2 < NCHK)
        def _():
            prepare_a(c0 + 2, jp, dpp)
        carry = advance(jp, dpp)
        reduce_b(c0 + 1)
        return carry

    lax.fori_loop(0, NCHK // 2, t_loop, carry0)

    pltpu.make_async_copy(out_a, out_hbm.at[pl.ds(0, VC * C)], out_sem_a).wait()
    pltpu.make_async_copy(out_b, out_hbm.at[pl.ds(0, VC * C)], out_sem_b).wait()


@jax.jit
def _spatial_transform(vol_tab, tx, ty, tz):
    mesh = plsc.VectorSubcoreMesh(core_axis_name="c", subcore_axis_name="s")
    run = pl.kernel(
        _sc_body,
        out_type=jax.ShapeDtypeStruct((NVOX * C,), jnp.float32),
        mesh=mesh,
        scratch_types=[
            pltpu.VMEM((VC,), jnp.float32),
            pltpu.VMEM((VC,), jnp.float32),
            pltpu.VMEM((VC,), jnp.float32),
            pltpu.VMEM((VC,), jnp.float32),
            pltpu.VMEM((VC,), jnp.float32),
            pltpu.VMEM((VC,), jnp.float32),
            pltpu.VMEM((NIDX,), jnp.int32),
            pltpu.VMEM((NIDX,), jnp.int32),
            pltpu.VMEM((8, VC), jnp.float32),
            pltpu.VMEM((8, VC), jnp.float32),
            pltpu.VMEM((NIDX, 16), jnp.float32),
            pltpu.VMEM((NIDX, 16), jnp.float32),
            pltpu.VMEM((VC * C,), jnp.float32),
            pltpu.VMEM((VC * C,), jnp.float32),
            pltpu.SemaphoreType.DMA,
            pltpu.SemaphoreType.DMA,
            pltpu.SemaphoreType.DMA,
            pltpu.SemaphoreType.DMA,
            pltpu.SemaphoreType.DMA,
            pltpu.SemaphoreType.DMA,
        ],
        compiler_params=pltpu.CompilerParams(
            needs_layout_passes=False,
            use_tc_tiling_on_sc=False,
        ),
    )
    return run(vol_tab, tx, ty, tz)


def _vol_pair(vt3):
    return pl.pallas_call(
        _vol_pair_body,
        grid=(NCOL // RB,),
        in_specs=[pl.BlockSpec((RB, C, D), lambda g: (g, 0, 0))],
        out_specs=pl.BlockSpec((RB, D * 16), lambda g: (g, 0)),
        out_shape=jax.ShapeDtypeStruct((NCOL, D * 16), jnp.float32),
    )(vt3)


def _out_back(o2):
    return pl.pallas_call(
        _out_back_body,
        grid=(NCOL // RB,),
        in_specs=[pl.BlockSpec((RB, D * C), lambda g: (g, 0))],
        out_specs=pl.BlockSpec((RB, C, D), lambda g: (g, 0, 0)),
        out_shape=jax.ShapeDtypeStruct((NCOL, C, D), jnp.float32),
    )(o2)


def kernel(vol, trf):
    vt3 = vol.transpose(0, 1, 2, 4, 3).reshape(NCOL, C, D)
    tt = trf.transpose(0, 1, 4, 2, 3).reshape(B * D, 3, D, D)
    tx = tt[:, 0].reshape(NVOX)
    ty = tt[:, 1].reshape(NVOX)
    tz = tt[:, 2].reshape(NVOX)
    vol_tab = _vol_pair(vt3).reshape(NVOX, 16)
    out1d = _spatial_transform(vol_tab, tx, ty, tz)
    o3 = _out_back(out1d.reshape(NCOL, D * C))
    return o3.reshape(B, D, D, C, D).transpose(0, 1, 2, 4, 3)


# speedup vs baseline: 4.5931x; 1.2314x over previous
import numpy as np

import jax
import jax.numpy as jnp
from jax import lax
from jax.experimental import pallas as pl
from jax.experimental.pallas import tpu as pltpu
from jax.experimental.pallas import tpu_sc as plsc

D = 96
C = 8
B = 2
PLANE = D * D
NVOX = B * D * D * D
NCOL = B * D * D
NW = 32
PPW = (B * D) // NW
VPW = NVOX // NW
CH = 6
VC = CH * D
NCHK = VPW // VC
NIDX = 4 * VC
NLIST = NIDX // 128
GPR = D // 16
RB = 128
OROW = NCOL * C


def _pair_select() -> np.ndarray:
    q = np.zeros((C, D, D * 16), np.float32)
    for k in range(D):
        for s in (0, 1):
            for c in range(C):
                q[c, min(k + s, D - 1), k * 16 + s * 8 + c] = 1.0
    return q.reshape(C * D, D * 16)


_Q = _pair_select()


def _vol_pair_body(x_ref, q_ref, o_ref):
    acc = None
    for c in range(C):
        part = jax.lax.dot(
            x_ref[:, c, :], q_ref[pl.ds(c * D, D), :],
            precision=lax.Precision.HIGHEST,
            preferred_element_type=jnp.float32)
        acc = part if acc is None else acc + part
    o_ref[...] = acc


def _sc_body(vol_hbm, trf_hbm, out_hbm,
             tx_a, ty_a, tz_a, tx_b, ty_b, tz_b,
             idx_a, idx_b, w_a, w_b, rows_a, rows_b, out_a, out_b,
             trf_sem_a, trf_sem_b, gat_sem_a, gat_sem_b,
             out_sem_a, out_sem_b):
    wid = lax.axis_index("s") * 2 + lax.axis_index("c")
    tile_base = wid * VPW
    lane = lax.iota(jnp.int32, 16)
    lane_f = lane.astype(jnp.float32)

    def make_prepare(tx_r, ty_r, tz_r, idx_r, rows_r, w_r, trf_sem, gat_sem):
        def prepare(x, jp, dpp):
            plane = wid * PPW + dpp
            for d, t_r in ((0, tx_r), (1, ty_r), (2, tz_r)):
                pltpu.make_async_copy(
                    trf_hbm.at[plane * 3 + d, pl.ds(jp, CH), :],
                    t_r, trf_sem).wait()
            b = jnp.where(plane >= D, 1, 0)
            i_f = (plane - D * b).astype(jnp.float32)
            vol_b_off = b * (D * PLANE)

            def row(r, _):
                j_f = (jp + r).astype(jnp.float32)
                voff0 = D * r
                for gr in range(GPR):
                    voff = voff0 + 16 * gr
                    tx = tx_r[r, pl.ds(16 * gr, 16)]
                    ty = ty_r[r, pl.ds(16 * gr, 16)]
                    tz = tz_r[r, pl.ds(16 * gr, 16)]
                    ci = jnp.minimum(jnp.maximum(i_f + tx, 0.0), float(D - 1))
                    cj = jnp.minimum(jnp.maximum(j_f + ty, 0.0), float(D - 1))
                    ck = jnp.minimum(jnp.maximum(
                        (16.0 * gr) + lane_f + tz, 0.0), float(D - 1))
                    bi = jnp.minimum(ci.astype(jnp.int32), D - 2)
                    bj = jnp.minimum(cj.astype(jnp.int32), D - 2)
                    bk = jnp.minimum(ck.astype(jnp.int32), D - 2)
                    wi1 = ci - bi.astype(jnp.float32)
                    wj1 = cj - bj.astype(jnp.float32)
                    wk1 = ck - bk.astype(jnp.float32)
                    wi0 = 1.0 - wi1
                    wj0 = 1.0 - wj1
                    wk0 = 1.0 - wk1
                    rb = vol_b_off + bi * PLANE + bj * D + bk
                    wlist = (wi0 * wj0, wi0 * wj1, wi1 * wj0, wi1 * wj1)
                    ibase = 4 * voff0 + 16 * gr
                    for cp, (di, dj) in enumerate(
                            ((0, 0), (0, 1), (1, 0), (1, 1))):
                        idx_r[pl.ds(ibase + D * cp, 16)] = (
                            rb + (di * PLANE + dj * D))
                        w_r[cp, pl.ds(voff, 16)] = wlist[cp] * wk0
                        w_r[cp + 4, pl.ds(voff, 16)] = wlist[cp] * wk1
                for lr in range(3):
                    off = 4 * voff0 + 128 * lr
                    pltpu.async_copy(
                        vol_hbm.at[idx_r.at[pl.ds(off, 128)]],
                        rows_r.at[pl.ds(off, 128)], gat_sem)
                return 0

            lax.fori_loop(0, CH, row, 0)

            @pl.when(x + 2 < NCHK)
            def _():
                nc = x + 2
                njp = jp + 2 * CH
                njp = jnp.where(njp >= D, njp - D, njp)
                nplane = wid * PPW + dpp + jnp.where(jp + 2 * CH >= D, 1, 0)
                for d, t_r in ((0, tx_r), (1, ty_r), (2, tz_r)):
                    pltpu.async_copy(
                        trf_hbm.at[nplane * 3 + d, pl.ds(njp, CH), :],
                        t_r, trf_sem)
        return prepare

    def make_reduce(idx_r, rows_r, w_r, out_r, gat_sem, out_sem):
        def reduce(c, jp, dpp):
            @pl.when(c >= 2)
            def _():
                pltpu.make_async_copy(
                    out_r, out_hbm.at[pl.ds(0, CH * C * 128)], out_sem).wait()

            def dr(l, _):
                off = 128 * l
                pltpu.make_async_copy(
                    vol_hbm.at[idx_r.at[pl.ds(off, 128)]],
                    rows_r.at[pl.ds(off, 128)], gat_sem).wait()
                return 0
            lax.fori_loop(0, NLIST, dr, 0)

            def row(r, _):
                voff0 = D * r
                rbase = r * (C * 128)
                for gr in range(GPR):
                    voff = voff0 + 16 * gr
                    ibase = 4 * voff0 + 16 * gr
                    rvecs = [jnp.full((16,), ibase + D * cp, jnp.int32) + lane
                             for cp in range(4)]
                    wv = [w_r[cp, pl.ds(voff, 16)] for cp in range(8)]
                    kv = jnp.full((16,), 16 * gr, jnp.int32) + lane
                    for ch in range(C):
                        chv = jnp.full((16,), ch, jnp.int32)
                        chv8 = jnp.full((16,), ch + 8, jnp.int32)
                        acc = wv[0] * plsc.load_gather(rows_r, [rvecs[0], chv])
                        for cp in range(1, 4):
                            acc = acc + wv[cp] * plsc.load_gather(
                                rows_r, [rvecs[cp], chv])
                        for cp in range(4):
                            acc = acc + wv[cp + 4] * plsc.load_gather(
                                rows_r, [rvecs[cp], chv8])
                        plsc.store_scatter(
                            out_r, [kv + (rbase + ch * 128)], acc)
                return 0
            lax.fori_loop(0, CH, row, 0)

            plane = wid * PPW + dpp
            obase = (plane * (D * C) + jp * C) * 128
            pltpu.async_copy(
                out_r, out_hbm.at[pl.ds(obase, CH * C * 128)], out_sem)
        return reduce

    prepare_a = make_prepare(tx_a, ty_a, tz_a, idx_a, rows_a, w_a,
                             trf_sem_a, gat_sem_a)
    prepare_b = make_prepare(tx_b, ty_b, tz_b, idx_b, rows_b, w_b,
                             trf_sem_b, gat_sem_b)
    reduce_a = make_reduce(idx_a, rows_a, w_a, out_a, gat_sem_a, out_sem_a)
    reduce_b = make_reduce(idx_b, rows_b, w_b, out_b, gat_sem_b, out_sem_b)

    def advance(jp, dpp):
        jp2 = jp + CH
        wrap = jp2 >= D
        return jnp.where(wrap, jp2 - D, jp2), dpp + wrap.astype(jnp.int32)

    plane0 = wid * PPW
    for d, t_r in ((0, tx_a), (1, ty_a), (2, tz_a)):
        pltpu.async_copy(
            trf_hbm.at[plane0 * 3 + d, pl.ds(0, CH), :], t_r, trf_sem_a)
    for d, t_r in ((0, tx_b), (1, ty_b), (2, tz_b)):
        pltpu.async_copy(
            trf_hbm.at[plane0 * 3 + d, pl.ds(CH, CH), :], t_r, trf_sem_b)
    z = jnp.int32(0)
    prepare_a(z, z, z)
    carry0 = advance(z, z)

    def t_loop(t, carry):
        jp1, dp1 = carry
        c0 = 2 * t
        jp0 = jnp.where(jp1 - CH < 0, jp1 - CH + D, jp1 - CH)
        dp0 = dp1 - jnp.where(jp1 - CH < 0, 1, 0)
        prepare_b(c0 + 1, jp1, dp1)
        jp2, dp2 = advance(jp1, dp1)
        reduce_a(c0, jp0, dp0)

        @pl.when(c0 + 2 < NCHK)
        def _():
            prepare_a(c0 + 2, jp2, dp2)
        reduce_b(c0 + 1, jp1, dp1)
        return advance(jp2, dp2)

    lax.fori_loop(0, NCHK // 2, t_loop, carry0)

    pltpu.make_async_copy(
        out_a, out_hbm.at[pl.ds(0, CH * C * 128)], out_sem_a).wait()
    pltpu.make_async_copy(
        out_b, out_hbm.at[pl.ds(0, CH * C * 128)], out_sem_b).wait()


@jax.jit
def _spatial_transform(vol_tab, trf3d):
    mesh = plsc.VectorSubcoreMesh(core_axis_name="c", subcore_axis_name="s")
    run = pl.kernel(
        _sc_body,
        out_type=jax.ShapeDtypeStruct((OROW * 128,), jnp.float32),
        mesh=mesh,
        scratch_types=[
            pltpu.VMEM((CH, 128), jnp.float32),
            pltpu.VMEM((CH, 128), jnp.float32),
            pltpu.VMEM((CH, 128), jnp.float32),
            pltpu.VMEM((CH, 128), jnp.float32),
            pltpu.VMEM((CH, 128), jnp.float32),
            pltpu.VMEM((CH, 128), jnp.float32),
            pltpu.VMEM((NIDX,), jnp.int32),
            pltpu.VMEM((NIDX,), jnp.int32),
            pltpu.VMEM((8, VC), jnp.float32),
            pltpu.VMEM((8, VC), jnp.float32),
            pltpu.VMEM((NIDX, 16), jnp.float32),
            pltpu.VMEM((NIDX, 16), jnp.float32),
            pltpu.VMEM((CH * C * 128,), jnp.float32),
            pltpu.VMEM((CH * C * 128,), jnp.float32),
            pltpu.SemaphoreType.DMA,
            pltpu.SemaphoreType.DMA,
            pltpu.SemaphoreType.DMA,
            pltpu.SemaphoreType.DMA,
            pltpu.SemaphoreType.DMA,
            pltpu.SemaphoreType.DMA,
        ],
        compiler_params=pltpu.CompilerParams(
            needs_layout_passes=False,
            use_tc_tiling_on_sc=False,
        ),
    )
    return run(vol_tab, trf3d)


def _vol_pair(vt3, q):
    return pl.pallas_call(
        _vol_pair_body,
        grid=(NCOL // RB,),
        in_specs=[
            pl.BlockSpec((RB, C, D), lambda g: (g, 0, 0)),
            pl.BlockSpec((C * D, D * 16), lambda g: (0, 0)),
        ],
        out_specs=pl.BlockSpec((RB, D * 16), lambda g: (g, 0)),
        out_shape=jax.ShapeDtypeStruct((NCOL, D * 16), jnp.float32),
    )(vt3, q)


def kernel(vol, trf):
    zero = lax.optimization_barrier(jnp.zeros((), jnp.float32))
    vt3 = vol.transpose(0, 1, 2, 4, 3).reshape(NCOL, C, D)
    tt = trf.transpose(0, 1, 4, 2, 3).reshape(B * D * 3, D, D)
    trf3d = jnp.pad(tt, ((0, 0), (0, 0), (0, 32)))
    vol_tab = _vol_pair(vt3, jnp.asarray(_Q)).reshape(NVOX, 16)
    outp = _spatial_transform(vol_tab, trf3d)
    o5 = outp.reshape(B, D, D, C, 128)[..., :D] + zero
    return o5.transpose(0, 1, 2, 4, 3)


# speedup vs baseline: 5.7829x; 1.2590x over previous
import numpy as np

import jax
import jax.numpy as jnp
from jax import lax
from jax.experimental import pallas as pl
from jax.experimental.pallas import tpu as pltpu
from jax.experimental.pallas import tpu_sc as plsc

D = 96
C = 8
B = 2
PLANE = D * D
NVOX = B * D * D * D
NCOL = B * D * D
NW = 32
PPW = (B * D) // NW
VPW = NVOX // NW
CH = 6
VC = CH * D
NCHK = VPW // VC
NIDX = 4 * VC
NLIST = NIDX // 128
GPR = D // 16
RB = 128
OROW = NCOL * C


def _pair_select() -> np.ndarray:
    q = np.zeros((C, D, D * 16), np.float32)
    for k in range(D):
        for s in (0, 1):
            for c in range(C):
                q[c, min(k + s, D - 1), k * 16 + s * 8 + c] = 1.0
    return q.reshape(C * D, D * 16)


_Q = _pair_select()


def _vol_pair_body(x_ref, q_ref, o_ref):
    acc = None
    for c in range(C):
        xc = x_ref[:, c, :]
        hi = xc.astype(jnp.bfloat16)
        lo = (xc - hi.astype(jnp.float32)).astype(jnp.bfloat16)
        qc = q_ref[pl.ds(c * D, D), :]
        part = jax.lax.dot(hi, qc, preferred_element_type=jnp.float32)
        part = part + jax.lax.dot(lo, qc, preferred_element_type=jnp.float32)
        acc = part if acc is None else acc + part
    o_ref[...] = acc


def _sc_body(vol_hbm, trf_hbm, out_hbm,
             tx_a, ty_a, tz_a, tx_b, ty_b, tz_b,
             idx_a, idx_b, w_a, w_b, rows_a, rows_b, out_a, out_b,
             trf_sem_a, trf_sem_b, gat_sem_a, gat_sem_b,
             out_sem_a, out_sem_b):
    wid = lax.axis_index("s") * 2 + lax.axis_index("c")
    tile_base = wid * VPW
    lane = lax.iota(jnp.int32, 16)
    lane_f = lane.astype(jnp.float32)

    def make_prepare(tx_r, ty_r, tz_r, idx_r, rows_r, w_r, trf_sem, gat_sem):
        def prepare(x, jp, dpp):
            plane = wid * PPW + dpp
            for d, t_r in ((0, tx_r), (1, ty_r), (2, tz_r)):
                pltpu.make_async_copy(
                    trf_hbm.at[plane * 3 + d, pl.ds(jp, CH), :],
                    t_r, trf_sem).wait()
            b = jnp.where(plane >= D, 1, 0)
            i_f = (plane - D * b).astype(jnp.float32)
            vol_b_off = b * (D * PLANE)

            def row(r, _):
                j_f = (jp + r).astype(jnp.float32)
                voff0 = D * r
                for gr in range(GPR):
                    voff = voff0 + 16 * gr
                    tx = tx_r[r, pl.ds(16 * gr, 16)]
                    ty = ty_r[r, pl.ds(16 * gr, 16)]
                    tz = tz_r[r, pl.ds(16 * gr, 16)]
                    ci = jnp.minimum(jnp.maximum(i_f + tx, 0.0), float(D - 1))
                    cj = jnp.minimum(jnp.maximum(j_f + ty, 0.0), float(D - 1))
                    ck = jnp.minimum(jnp.maximum(
                        (16.0 * gr) + lane_f + tz, 0.0), float(D - 1))
                    bi = jnp.minimum(ci.astype(jnp.int32), D - 2)
                    bj = jnp.minimum(cj.astype(jnp.int32), D - 2)
                    bk = jnp.minimum(ck.astype(jnp.int32), D - 2)
                    wi1 = ci - bi.astype(jnp.float32)
                    wj1 = cj - bj.astype(jnp.float32)
                    wk1 = ck - bk.astype(jnp.float32)
                    wi0 = 1.0 - wi1
                    wj0 = 1.0 - wj1
                    wk0 = 1.0 - wk1
                    rb = vol_b_off + bi * PLANE + bj * D + bk
                    wlist = (wi0 * wj0, wi0 * wj1, wi1 * wj0, wi1 * wj1)
                    ibase = 4 * voff0 + 16 * gr
                    for cp, (di, dj) in enumerate(
                            ((0, 0), (0, 1), (1, 0), (1, 1))):
                        idx_r[pl.ds(ibase + D * cp, 16)] = (
                            rb + (di * PLANE + dj * D))
                        w_r[cp, pl.ds(voff, 16)] = wlist[cp] * wk0
                        w_r[cp + 4, pl.ds(voff, 16)] = wlist[cp] * wk1
                for lr in range(3):
                    off = 4 * voff0 + 128 * lr
                    pltpu.async_copy(
                        vol_hbm.at[idx_r.at[pl.ds(off, 128)]],
                        rows_r.at[pl.ds(off, 128)], gat_sem)
                return 0

            lax.fori_loop(0, CH, row, 0)

            @pl.when(x + 2 < NCHK)
            def _():
                nc = x + 2
                njp = jp + 2 * CH
                njp = jnp.where(njp >= D, njp - D, njp)
                nplane = wid * PPW + dpp + jnp.where(jp + 2 * CH >= D, 1, 0)
                for d, t_r in ((0, tx_r), (1, ty_r), (2, tz_r)):
                    pltpu.async_copy(
                        trf_hbm.at[nplane * 3 + d, pl.ds(njp, CH), :],
                        t_r, trf_sem)
        return prepare

    def make_reduce(idx_r, rows_r, w_r, out_r, gat_sem, out_sem):
        def reduce(c, jp, dpp):
            @pl.when(c >= 2)
            def _():
                pltpu.make_async_copy(
                    out_r, out_hbm.at[pl.ds(0, CH * C * 128)], out_sem).wait()

            def dr(l, _):
                off = 128 * l
                pltpu.make_async_copy(
                    vol_hbm.at[idx_r.at[pl.ds(off, 128)]],
                    rows_r.at[pl.ds(off, 128)], gat_sem).wait()
                return 0
            lax.fori_loop(0, NLIST, dr, 0)

            def row(r, _):
                voff0 = D * r
                rbase = r * (C * 128)
                for gr in range(GPR):
                    voff = voff0 + 16 * gr
                    ibase = 4 * voff0 + 16 * gr
                    rvecs = [jnp.full((16,), ibase + D * cp, jnp.int32) + lane
                             for cp in range(4)]
                    wv = [w_r[cp, pl.ds(voff, 16)] for cp in range(8)]
                    kv = jnp.full((16,), 16 * gr, jnp.int32) + lane
                    for ch in range(C):
                        chv = jnp.full((16,), ch, jnp.int32)
                        chv8 = jnp.full((16,), ch + 8, jnp.int32)
                        acc = wv[0] * plsc.load_gather(rows_r, [rvecs[0], chv])
                        for cp in range(1, 4):
                            acc = acc + wv[cp] * plsc.load_gather(
                                rows_r, [rvecs[cp], chv])
                        for cp in range(4):
                            acc = acc + wv[cp + 4] * plsc.load_gather(
                                rows_r, [rvecs[cp], chv8])
                        plsc.store_scatter(
                            out_r, [kv + (rbase + ch * 128)], acc)
                return 0
            lax.fori_loop(0, CH, row, 0)

            plane = wid * PPW + dpp
            obase = (plane * (D * C) + jp * C) * 128
            pltpu.async_copy(
                out_r, out_hbm.at[pl.ds(obase, CH * C * 128)], out_sem)
        return reduce

    prepare_a = make_prepare(tx_a, ty_a, tz_a, idx_a, rows_a, w_a,
                             trf_sem_a, gat_sem_a)
    prepare_b = make_prepare(tx_b, ty_b, tz_b, idx_b, rows_b, w_b,
                             trf_sem_b, gat_sem_b)
    reduce_a = make_reduce(idx_a, rows_a, w_a, out_a, gat_sem_a, out_sem_a)
    reduce_b = make_reduce(idx_b, rows_b, w_b, out_b, gat_sem_b, out_sem_b)

    def advance(jp, dpp):
        jp2 = jp + CH
        wrap = jp2 >= D
        return jnp.where(wrap, jp2 - D, jp2), dpp + wrap.astype(jnp.int32)

    plane0 = wid * PPW
    for d, t_r in ((0, tx_a), (1, ty_a), (2, tz_a)):
        pltpu.async_copy(
            trf_hbm.at[plane0 * 3 + d, pl.ds(0, CH), :], t_r, trf_sem_a)
    for d, t_r in ((0, tx_b), (1, ty_b), (2, tz_b)):
        pltpu.async_copy(
            trf_hbm.at[plane0 * 3 + d, pl.ds(CH, CH), :], t_r, trf_sem_b)
    z = jnp.int32(0)
    prepare_a(z, z, z)
    carry0 = advance(z, z)

    def t_loop(t, carry):
        jp1, dp1 = carry
        c0 = 2 * t
        jp0 = jnp.where(jp1 - CH < 0, jp1 - CH + D, jp1 - CH)
        dp0 = dp1 - jnp.where(jp1 - CH < 0, 1, 0)
        prepare_b(c0 + 1, jp1, dp1)
        jp2, dp2 = advance(jp1, dp1)
        reduce_a(c0, jp0, dp0)

        @pl.when(c0 + 2 < NCHK)
        def _():
            prepare_a(c0 + 2, jp2, dp2)
        reduce_b(c0 + 1, jp1, dp1)
        return advance(jp2, dp2)

    lax.fori_loop(0, NCHK // 2, t_loop, carry0)

    pltpu.make_async_copy(
        out_a, out_hbm.at[pl.ds(0, CH * C * 128)], out_sem_a).wait()
    pltpu.make_async_copy(
        out_b, out_hbm.at[pl.ds(0, CH * C * 128)], out_sem_b).wait()


@jax.jit
def _spatial_transform(vol_tab, trf3d):
    mesh = plsc.VectorSubcoreMesh(core_axis_name="c", subcore_axis_name="s")
    run = pl.kernel(
        _sc_body,
        out_type=jax.ShapeDtypeStruct((OROW * 128,), jnp.float32),
        mesh=mesh,
        scratch_types=[
            pltpu.VMEM((CH, 128), jnp.float32),
            pltpu.VMEM((CH, 128), jnp.float32),
            pltpu.VMEM((CH, 128), jnp.float32),
            pltpu.VMEM((CH, 128), jnp.float32),
            pltpu.VMEM((CH, 128), jnp.float32),
            pltpu.VMEM((CH, 128), jnp.float32),
            pltpu.VMEM((NIDX,), jnp.int32),
            pltpu.VMEM((NIDX,), jnp.int32),
            pltpu.VMEM((8, VC), jnp.float32),
            pltpu.VMEM((8, VC), jnp.float32),
            pltpu.VMEM((NIDX, 16), jnp.float32),
            pltpu.VMEM((NIDX, 16), jnp.float32),
            pltpu.VMEM((CH * C * 128,), jnp.float32),
            pltpu.VMEM((CH * C * 128,), jnp.float32),
            pltpu.SemaphoreType.DMA,
            pltpu.SemaphoreType.DMA,
            pltpu.SemaphoreType.DMA,
            pltpu.SemaphoreType.DMA,
            pltpu.SemaphoreType.DMA,
            pltpu.SemaphoreType.DMA,
        ],
        compiler_params=pltpu.CompilerParams(
            needs_layout_passes=False,
            use_tc_tiling_on_sc=False,
        ),
    )
    return run(vol_tab, trf3d)


def _vol_pair(vt3, q):
    return pl.pallas_call(
        _vol_pair_body,
        grid=(NCOL // RB,),
        in_specs=[
            pl.BlockSpec((RB, C, D), lambda g: (g, 0, 0)),
            pl.BlockSpec((C * D, D * 16), lambda g: (0, 0)),
        ],
        out_specs=pl.BlockSpec((RB, D * 16), lambda g: (g, 0)),
        out_shape=jax.ShapeDtypeStruct((NCOL, D * 16), jnp.float32),
    )(vt3, q)


def kernel(vol, trf):
    zero = lax.optimization_barrier(jnp.zeros((), jnp.float32))
    vt3 = vol.transpose(0, 1, 2, 4, 3).reshape(NCOL, C, D)
    tt = trf.transpose(0, 1, 4, 2, 3).reshape(B * D * 3, D, D)
    trf3d = jnp.pad(tt, ((0, 0), (0, 0), (0, 32)))
    vol_tab = _vol_pair(vt3, jnp.asarray(_Q, dtype=jnp.bfloat16)).reshape(NVOX, 16)
    outp = _spatial_transform(vol_tab, trf3d)
    o5 = outp.reshape(B, D, D, C, 128)[..., :D] + zero
    return o5.transpose(0, 1, 2, 4, 3)
